# baseline (speedup 1.0000x reference)
import functools

import jax
import jax.numpy as jnp
from jax import lax
from jax.experimental import pallas as pl
from jax.experimental.pallas import tpu as pltpu
from jax.experimental.pallas import tpu_sc as plsc

B = 4096
BX = 64
BY = 64
NKEY = BX * BY
NWORDS = NKEY * B

NC = 2
NS = 16
NW = NC * NS
KPW = NKEY // NW
WPW = KPW * B

CW = 32768
NCH = WPW // CW

_mesh = plsc.VectorSubcoreMesh(
    core_axis_name="c", subcore_axis_name="s", num_cores=NC, num_subcores=NS
)


@functools.partial(
    pl.kernel,
    out_type=jax.ShapeDtypeStruct((NWORDS,), jnp.int32),
    mesh=_mesh,
    compiler_params=pltpu.CompilerParams(needs_layout_passes=False),
    scratch_types=[
        pltpu.VMEM((B * 2,), jnp.int32),
        pltpu.VMEM((B + 16,), jnp.int32),
        pltpu.VMEM((CW,), jnp.int32),
        pltpu.VMEM((CW,), jnp.int32),
        pltpu.SemaphoreType.DMA,
        pltpu.SemaphoreType.DMA,
    ],
)
def _onehot2d_sc(loc_hbm, out_hbm, loc_v, ml_v, buf0, buf1, sem0, sem1):
    wid = lax.axis_index("s") * NC + lax.axis_index("c")
    klo = wid * KPW
    wbase = wid * WPW

    pltpu.sync_copy(loc_hbm, loc_v)

    iota = lax.iota(jnp.int32, 16)
    zv = jnp.zeros((16,), jnp.int32)
    ones = jnp.full((16,), 1, jnp.int32)

    def _zero(buf):
        def body(i, c):
            buf[pl.ds(i * 16, 16)] = zv
            return c
        lax.fori_loop(0, CW // 16, body, 0)

    _zero(buf0)
    _zero(buf1)

    def _scan(i, off):
        bvec = iota + i * 16
        xv = plsc.load_gather(loc_v, [bvec * 2])
        yv = plsc.load_gather(loc_v, [bvec * 2 + 1])
        key = xv * BY + yv
        m = (key >= klo) & (key < klo + KPW)
        mw = (key - klo) * B + bvec
        plsc.store_compressed(ml_v.at[pl.ds(off, 16)], mw, mask=m)
        cnt = plsc.all_reduce_population_count(m)
        return off + cnt[0]

    nmatch = lax.fori_loop(0, B // 16, _scan, 0)
    ml_v[pl.ds(nmatch, 16)] = jnp.full((16,), -1, jnp.int32)
    ngrp = (nmatch + 15) // 16

    def _paint(buf, lo, val):
        def body(i, c):
            mv = ml_v[pl.ds(i * 16, 16)]
            m = (mv >= lo) & (mv < lo + CW)
            idx = lax.select(m, mv - lo, zv)
            plsc.store_scatter(buf, [idx], val, mask=m)
            return c
        lax.fori_loop(0, ngrp, body, 0)

    bufs = (buf0, buf1)
    sems = (sem0, sem1)
    copies = [None] * NCH
    for c in range(NCH):
        buf = bufs[c % 2]
        if c >= 2:
            copies[c - 2].wait()
            _paint(buf, (c - 2) * CW, zv)
        _paint(buf, c * CW, ones)
        copies[c] = pltpu.async_copy(
            buf, out_hbm.at[pl.ds(wbase + c * CW, CW)], sems[c % 2]
        )
    copies[NCH - 2].wait()
    copies[NCH - 1].wait()


def kernel(loc):
    flat = _onehot2d_sc(loc.reshape(-1))
    return flat.reshape(BX, BY, B).transpose(2, 0, 1)[..., None]


# speedup vs baseline: 3.3056x; 3.3056x over previous
import functools

import jax
import jax.numpy as jnp
from jax import lax
from jax.experimental import pallas as pl
from jax.experimental.pallas import tpu as pltpu
from jax.experimental.pallas import tpu_sc as plsc

B = 4096
BX = 64
BY = 64
NKEY = BX * BY
NWORDS = NKEY * B

NC = 2
NS = 16
NW = NC * NS
KPW = NKEY // NW
WPW = KPW * B

CW = 32768
NCH = WPW // CW

_mesh = plsc.VectorSubcoreMesh(
    core_axis_name="c", subcore_axis_name="s", num_cores=NC, num_subcores=NS
)


@functools.partial(
    pl.kernel,
    out_type=jax.ShapeDtypeStruct((NWORDS,), jnp.int32),
    mesh=_mesh,
    compiler_params=pltpu.CompilerParams(needs_layout_passes=False),
    scratch_types=[
        pltpu.VMEM((B * 2,), jnp.int32),
        pltpu.VMEM((B + 16,), jnp.int32),
        pltpu.VMEM((CW,), jnp.int32),
        pltpu.VMEM((CW,), jnp.int32),
        pltpu.SemaphoreType.DMA,
        pltpu.SemaphoreType.DMA,
    ],
)
def _onehot2d_sc(loc_hbm, out_hbm, loc_v, ml_v, buf0, buf1, sem0, sem1):
    wid = lax.axis_index("s") * NC + lax.axis_index("c")
    klo = wid * KPW
    wbase = wid * WPW

    pltpu.sync_copy(loc_hbm, loc_v)

    iota = lax.iota(jnp.int32, 16)
    zv = jnp.zeros((16,), jnp.int32)
    ones = jnp.full((16,), 1, jnp.int32)

    def _zero(buf):
        def body(i, c):
            buf[pl.ds(i * 16, 16)] = zv
            return c
        lax.fori_loop(0, CW // 16, body, 0)

    _zero(buf0)
    _zero(buf1)

    def _scan(i, off):
        bvec = iota + i * 16
        xv = plsc.load_gather(loc_v, [bvec * 2])
        yv = plsc.load_gather(loc_v, [bvec * 2 + 1])
        key = xv * BY + yv
        m = (key >= klo) & (key < klo + KPW)
        mw = (key - klo) * B + bvec
        plsc.store_compressed(ml_v.at[pl.ds(off, 16)], mw, mask=m)
        cnt = plsc.all_reduce_population_count(m)
        return off + cnt[0]

    nmatch = lax.fori_loop(0, B // 16, _scan, 0)
    ml_v[pl.ds(nmatch, 16)] = jnp.full((16,), -1, jnp.int32)
    ngrp = (nmatch + 15) // 16

    def _paint(buf, lo, val):
        def body(i, c):
            mv = ml_v[pl.ds(i * 16, 16)]
            m = (mv >= lo) & (mv < lo + CW)
            idx = lax.select(m, mv - lo, zv)
            plsc.store_scatter(buf, [idx], val, mask=m)
            return c
        lax.fori_loop(0, ngrp, body, 0)

    bufs = (buf0, buf1)
    sems = (sem0, sem1)
    copies = [None] * NCH
    for c in range(NCH):
        buf = bufs[c % 2]
        if c >= 2:
            copies[c - 2].wait()
            _paint(buf, (c - 2) * CW, zv)
        _paint(buf, c * CW, ones)
        copies[c] = pltpu.async_copy(
            buf, out_hbm.at[pl.ds(wbase + c * CW, CW)], sems[c % 2]
        )
    copies[NCH - 2].wait()
    copies[NCH - 1].wait()


def kernel(loc):
    flat = _onehot2d_sc(loc.reshape(-1))
    return flat.reshape(BX, BY, B, 1).transpose(2, 0, 1, 3)


# speedup vs baseline: 4.4590x; 1.3489x over previous
import functools

import jax
import jax.numpy as jnp
from jax import lax
from jax.experimental import pallas as pl
from jax.experimental.pallas import tpu as pltpu
from jax.experimental.pallas import tpu_sc as plsc

B = 4096
BX = 64
BY = 64
NKEY = BX * BY
NWORDS = NKEY * B

NC = 2
NS = 16
NW = NC * NS
KPW = NKEY // NW
WPW = KPW * B

NBUF = 4
CW = 16384
NCH = WPW // CW

_mesh = plsc.VectorSubcoreMesh(
    core_axis_name="c", subcore_axis_name="s", num_cores=NC, num_subcores=NS
)


@functools.partial(
    pl.kernel,
    out_type=jax.ShapeDtypeStruct((NWORDS,), jnp.int32),
    mesh=_mesh,
    compiler_params=pltpu.CompilerParams(needs_layout_passes=False),
    scratch_types=[
        pltpu.VMEM((B * 2,), jnp.int32),
        pltpu.VMEM((B + 16,), jnp.int32),
        *[pltpu.VMEM((CW,), jnp.int32) for _ in range(NBUF)],
        *[pltpu.SemaphoreType.DMA for _ in range(NBUF)],
    ],
)
def _onehot2d_sc(xy_hbm, out_hbm, xy_v, ml_v, *bufsems):
    bufs = bufsems[:NBUF]
    sems = bufsems[NBUF:]
    wid = lax.axis_index("s") * NC + lax.axis_index("c")
    klo = wid * KPW
    wbase = wid * WPW

    pltpu.sync_copy(xy_hbm, xy_v)

    iota = lax.iota(jnp.int32, 16)
    zv = jnp.zeros((16,), jnp.int32)
    ones = jnp.full((16,), 1, jnp.int32)

    def _scan(i, off):
        xv = xy_v[pl.ds(i * 16, 16)]
        yv = xy_v[pl.ds(B + i * 16, 16)]
        key = xv * BY + yv
        m = (key >= klo) & (key < klo + KPW)
        mw = (key - klo) * B + iota + i * 16
        plsc.store_compressed(ml_v.at[pl.ds(off, 16)], mw, mask=m)
        cnt = plsc.all_reduce_population_count(m)
        return off + cnt[0]

    nmatch = lax.fori_loop(0, B // 16, _scan, 0)
    ml_v[pl.ds(nmatch, 16)] = jnp.full((16,), -1, jnp.int32)
    ngrp = (nmatch + 15) // 16

    def _zero(buf):
        def body(i, c):
            for u in range(8):
                buf[pl.ds(i * 128 + u * 16, 16)] = zv
            return c
        lax.fori_loop(0, CW // 128, body, 0)

    def _paint(buf, lo, val):
        def body(i, c):
            mv = ml_v[pl.ds(i * 16, 16)]
            m = (mv >= lo) & (mv < lo + CW)
            idx = lax.select(m, mv - lo, zv)
            plsc.store_scatter(buf, [idx], val, mask=m)
            return c
        lax.fori_loop(0, ngrp, body, 0)

    copies = [None] * NCH
    for c in range(NCH):
        buf = bufs[c % NBUF]
        if c < NBUF:
            _zero(buf)
        else:
            copies[c - NBUF].wait()
            _paint(buf, (c - NBUF) * CW, zv)
        _paint(buf, c * CW, ones)
        copies[c] = pltpu.async_copy(
            buf, out_hbm.at[pl.ds(wbase + c * CW, CW)], sems[c % NBUF]
        )
    for c in range(NCH - NBUF, NCH):
        copies[c].wait()


def kernel(loc):
    flat = _onehot2d_sc(loc.T.reshape(-1))
    return flat.reshape(BX, BY, B, 1).transpose(2, 0, 1, 3)


# speedup vs baseline: 4.6441x; 1.0415x over previous
import functools

import jax
import jax.numpy as jnp
from jax import lax
from jax.experimental import pallas as pl
from jax.experimental.pallas import tpu as pltpu
from jax.experimental.pallas import tpu_sc as plsc

B = 4096
BX = 64
BY = 64
NKEY = BX * BY
NWORDS = NKEY * B

NC = 2
NS = 16
NW = NC * NS
KPW = NKEY // NW
WPW = KPW * B

NBUF = 4
CW = 16384
NCH = WPW // CW
NGRP = NCH // NBUF

_mesh = plsc.VectorSubcoreMesh(
    core_axis_name="c", subcore_axis_name="s", num_cores=NC, num_subcores=NS
)


@functools.partial(
    pl.kernel,
    out_type=jax.ShapeDtypeStruct((NWORDS,), jnp.int32),
    mesh=_mesh,
    compiler_params=pltpu.CompilerParams(needs_layout_passes=False),
    scratch_types=[
        pltpu.VMEM((B * 2,), jnp.int32),
        pltpu.VMEM((B + 16,), jnp.int32),
        *[pltpu.VMEM((CW,), jnp.int32) for _ in range(NBUF)],
        *[pltpu.SemaphoreType.DMA for _ in range(NBUF)],
        pltpu.SemaphoreType.DMA,
    ],
)
def _onehot2d_sc(xy_hbm, out_hbm, xy_v, ml_v, *bufsems):
    bufs = bufsems[:NBUF]
    sems = bufsems[NBUF:2 * NBUF]
    lsem = bufsems[2 * NBUF]
    wid = lax.axis_index("s") * NC + lax.axis_index("c")
    klo = wid * KPW
    wbase = wid * WPW

    iota = lax.iota(jnp.int32, 16)
    zv = jnp.zeros((16,), jnp.int32)
    ones = jnp.full((16,), 1, jnp.int32)

    stage = pltpu.async_copy(xy_hbm, xy_v, lsem)

    def _zero(buf):
        def body(i, c):
            for u in range(8):
                buf[pl.ds(i * 128 + u * 16, 16)] = zv
            return c
        lax.fori_loop(0, CW // 128, body, 0)

    for buf in bufs:
        _zero(buf)
    stage.wait()

    def _scan(i, off):
        base = i * 16 + (i // 8) * 128
        xv = xy_v[pl.ds(base, 16)]
        yv = xy_v[pl.ds(base + 128, 16)]
        key = xv * BY + yv
        m = (key >= klo) & (key < klo + KPW)
        mw = (key - klo) * B + iota + i * 16
        plsc.store_compressed(ml_v.at[pl.ds(off, 16)], mw, mask=m)
        cnt = plsc.all_reduce_population_count(m)
        return off + cnt[0]

    nmatch = lax.fori_loop(0, B // 16, _scan, 0)
    ml_v[pl.ds(nmatch, 16)] = jnp.full((16,), -1, jnp.int32)
    ngrp = (nmatch + 15) // 16

    def _paint(buf, lo, val):
        def body(i, c):
            mv = ml_v[pl.ds(i * 16, 16)]
            m = (mv >= lo) & (mv < lo + CW)
            idx = lax.select(m, mv - lo, zv)
            plsc.store_scatter(buf, [idx], val, mask=m)
            return c
        lax.fori_loop(0, ngrp, body, 0)

    def _fire(buf, lo, sem):
        pltpu.async_copy(buf, out_hbm.at[pl.ds(wbase + lo, CW)], sem)

    def _drain(buf, sem):
        pltpu.make_async_copy(buf, out_hbm.at[pl.ds(wbase, CW)], sem).wait()

    for u in range(NBUF):
        _paint(bufs[u], u * CW, ones)
        _fire(bufs[u], u * CW, sems[u])

    def _group(g, c):
        for u in range(NBUF):
            lo = (g * NBUF + u) * CW
            _drain(bufs[u], sems[u])
            _paint(bufs[u], lo - NBUF * CW, zv)
            _paint(bufs[u], lo, ones)
            _fire(bufs[u], lo, sems[u])
        return c

    lax.fori_loop(1, NGRP, _group, 0)

    for u in range(NBUF):
        _drain(bufs[u], sems[u])


def kernel(loc):
    xy = loc.reshape(32, 128, 2).transpose(0, 2, 1).reshape(-1)
    flat = _onehot2d_sc(xy)
    return flat.reshape(BX, BY, B, 1).transpose(2, 0, 1, 3)
